# Initial kernel scaffold; baseline (speedup 1.0000x reference)
import jax
import jax.numpy as jnp
from jax.experimental import pallas as pl


def kernel(input_ids, token_table, pos_table):
    raise NotImplementedError("write your pallas kernel here")


# speedup vs baseline: 18.0279x; 18.0279x over previous
import functools

import jax
import jax.numpy as jnp
from jax import lax
from jax.experimental import pallas as pl
from jax.experimental.pallas import tpu as pltpu
from jax.experimental.pallas import tpu_sc as plsc

NC = 2
NS = 16
NW = NC * NS
HALF = 100


def _build_call(B, S, V, H):
    CHUNK = S
    rows_per_w = B * S // NW
    n_chunks = rows_per_w // CHUNK
    mesh = plsc.VectorSubcoreMesh(core_axis_name="c", subcore_axis_name="s")

    def body(idx_hbm, tok_hbm, pos_hbm, out_hbm,
             idx_v, pos_v, d0, d1, g0, g1, s0, s1):
        sid = lax.axis_index("s")
        wid = sid * NC + lax.axis_index("c")
        base = wid * rows_per_w

        pltpu.sync_copy(idx_hbm.at[wid], idx_v)

        @pl.when(sid == 0)
        def _():
            pltpu.sync_copy(pos_hbm, pos_v)
        plsc.subcore_barrier()

        bufs = (d0, d1)
        gsems = (g0, g1)
        ssems = (s0, s1)

        def gather(j, b, wait):
            d = bufs[b]
            for h in range(2):
                cp = pltpu.make_async_copy(
                    tok_hbm.at[idx_v.at[2 * j + h]],
                    d.at[pl.ds(h * HALF, HALF)],
                    gsems[b])
                cp.wait() if wait else cp.start(add=True)

        def scatter(j, b, wait):
            cp = pltpu.make_async_copy(
                bufs[b], out_hbm.at[pl.ds(base + j * CHUNK, CHUNK)], ssems[b])
            cp.wait() if wait else cp.start()

        for b in range(2):
            pltpu.sync_copy(pos_v, bufs[b])
            gather(b, b, wait=False)

        def step(k, _):
            for b in range(2):
                j = 2 * k + b
                gather(j, b, wait=True)
                scatter(j, b, wait=False)
                scatter(j, b, wait=True)
                pltpu.sync_copy(pos_v, bufs[b])
                gather(j + 2, b, wait=False)
            return 0

        lax.fori_loop(0, n_chunks // 2 - 1, step, 0)

        for b in range(2):
            j = n_chunks - 2 + b
            gather(j, b, wait=True)
            scatter(j, b, wait=False)
            scatter(j, b, wait=True)

    return pl.kernel(
        body,
        out_type=jax.ShapeDtypeStruct((B * S, H), jnp.float32),
        mesh=mesh,
        scratch_types=[
            pltpu.VMEM((2 * n_chunks, HALF), jnp.int32),
            pltpu.VMEM_SHARED((CHUNK, H), jnp.float32),
            pltpu.VMEM((CHUNK, H), jnp.float32),
            pltpu.VMEM((CHUNK, H), jnp.float32),
            pltpu.SemaphoreType.DMA,
            pltpu.SemaphoreType.DMA,
            pltpu.SemaphoreType.DMA,
            pltpu.SemaphoreType.DMA,
        ],
    )


@jax.jit
def kernel(input_ids, token_table, pos_table):
    B, S = input_ids.shape
    V, H = token_table.shape
    idx_r = input_ids.reshape(NW, -1, HALF).astype(jnp.int32)
    pos_s = pos_table[:S]
    out = _build_call(B, S, V, H)(idx_r, token_table, pos_s)
    return out.reshape(B, S, H)
